# baseline (speedup 1.0000x reference)
import jax
import jax.numpy as jnp
from jax.experimental import pallas as pl

BATCH_BLOCK = 64


def _add_kernel(x_ref, pos_ref, out_ref):
    out_ref[...] = x_ref[...] + pos_ref[...]


def kernel(x, pos_table):
    batch, maxlen, dim = x.shape
    grid = (batch // BATCH_BLOCK,)
    return pl.pallas_call(
        _add_kernel,
        grid=grid,
        in_specs=[
            pl.BlockSpec((BATCH_BLOCK, maxlen, dim), lambda i: (i, 0, 0)),
            pl.BlockSpec((1, maxlen, dim), lambda i: (0, 0, 0)),
        ],
        out_specs=pl.BlockSpec((BATCH_BLOCK, maxlen, dim), lambda i: (i, 0, 0)),
        out_shape=jax.ShapeDtypeStruct((batch, maxlen, dim), x.dtype),
    )(x, pos_table[None])


# speedup vs baseline: 1.6645x; 1.6645x over previous
import jax
import jax.numpy as jnp
from jax.experimental import pallas as pl

BATCH_BLOCK = 128


def _add_kernel(x_ref, pos_ref, out_ref):
    out_ref[...] = x_ref[...] + pos_ref[...]


def kernel(x, pos_table):
    batch, maxlen, dim = x.shape
    x2 = x.reshape(batch, maxlen * dim)
    pos2 = pos_table.reshape(1, maxlen * dim)
    grid = (batch // BATCH_BLOCK,)
    out = pl.pallas_call(
        _add_kernel,
        grid=grid,
        in_specs=[
            pl.BlockSpec((BATCH_BLOCK, maxlen * dim), lambda i: (i, 0)),
            pl.BlockSpec((1, maxlen * dim), lambda i: (0, 0)),
        ],
        out_specs=pl.BlockSpec((BATCH_BLOCK, maxlen * dim), lambda i: (i, 0)),
        out_shape=jax.ShapeDtypeStruct((batch, maxlen * dim), x.dtype),
    )(x2, pos2)
    return out.reshape(batch, maxlen, dim)


# speedup vs baseline: 1.6649x; 1.0002x over previous
import jax
import jax.numpy as jnp
from jax.experimental import pallas as pl
from jax.experimental.pallas import tpu as pltpu

BATCH_BLOCK = 128


def _add_kernel(x_ref, pos_ref, out_ref):
    out_ref[...] = x_ref[...] + pos_ref[...]


def kernel(x, pos_table):
    batch, maxlen, dim = x.shape
    x2 = x.reshape(batch, maxlen * dim)
    pos2 = pos_table.reshape(1, maxlen * dim)
    grid = (batch // BATCH_BLOCK,)
    out = pl.pallas_call(
        _add_kernel,
        grid=grid,
        in_specs=[
            pl.BlockSpec((BATCH_BLOCK, maxlen * dim), lambda i: (i, 0)),
            pl.BlockSpec((1, maxlen * dim), lambda i: (0, 0)),
        ],
        out_specs=pl.BlockSpec((BATCH_BLOCK, maxlen * dim), lambda i: (i, 0)),
        out_shape=jax.ShapeDtypeStruct((batch, maxlen * dim), x.dtype),
        compiler_params=pltpu.CompilerParams(
            dimension_semantics=("parallel",)),
    )(x2, pos2)
    return out.reshape(batch, maxlen, dim)
